# baseline (speedup 1.0000x reference)
import functools

import jax
import jax.numpy as jnp
from jax import lax
from jax.experimental import pallas as pl
from jax.experimental.pallas import tpu as pltpu
from jax.experimental.pallas import tpu_sc as plsc

EMBED = 64
NUM_CLASSES = 32
BATCH = 4096
SEQ = 200

NC = 2
NS = 16
NW = NC * NS
BPW = BATCH // NW
CHUNK = 100
NCHUNK = SEQ // CHUNK
LANES = 16
NVREG = EMBED // LANES


def _sc_pool(x_r, table):
    mesh = plsc.VectorSubcoreMesh(core_axis_name="c", subcore_axis_name="s")

    @functools.partial(
        pl.kernel,
        out_type=jax.ShapeDtypeStruct((NW, BPW, EMBED), jnp.float32),
        mesh=mesh,
        scratch_types=[
            pltpu.VMEM((BPW, NCHUNK, CHUNK), jnp.int32),
            pltpu.VMEM((2, SEQ, EMBED), jnp.float32),
            pltpu.VMEM((BPW, EMBED), jnp.float32),
            pltpu.SemaphoreType.DMA,
            pltpu.SemaphoreType.DMA,
        ],
        compiler_params=pltpu.CompilerParams(use_tc_tiling_on_sc=False),
    )
    def pool(x_hbm, table_hbm, out_hbm, idx_v, buf_v, acc_v, sem0, sem1):
        wid = lax.axis_index("s") * NC + lax.axis_index("c")
        sems = (sem0, sem1)
        pltpu.sync_copy(x_hbm.at[wid], idx_v)

        def issue(b, p):
            for j in range(NCHUNK):
                pltpu.async_copy(
                    table_hbm.at[idx_v.at[b, j]],
                    buf_v.at[p, pl.ds(j * CHUNK, CHUNK)],
                    sems[p],
                )

        def drain(p):
            pltpu.make_async_copy(
                table_hbm.at[pl.ds(0, SEQ)], buf_v.at[p], sems[p]
            ).wait()

        issue(0, 0)
        issue(1, 1)

        def outer(g, carry):
            for p in range(2):
                b = g * 2 + p
                drain(p)

                @pl.when(b + 2 < BPW)
                def _():
                    issue(b + 2, p)

                zero = jnp.zeros((LANES,), jnp.float32)

                def inner(i, accs):
                    out = list(accs)
                    for u in range(4):
                        r = i * 4 + u
                        s = (u % 2) * NVREG
                        for k in range(NVREG):
                            out[s + k] = out[s + k] + buf_v[p, r, pl.ds(LANES * k, LANES)]
                    return tuple(out)

                accs = lax.fori_loop(0, SEQ // 4, inner, (zero,) * (2 * NVREG))
                for k in range(NVREG):
                    acc_v[b, pl.ds(LANES * k, LANES)] = accs[k] + accs[NVREG + k]
            return carry

        lax.fori_loop(0, BPW // 2, outer, 0)
        pltpu.sync_copy(acc_v, out_hbm.at[wid])

    return pool(x_r, table)


def _tc_head(sums, wt, bias):
    def head(s_ref, w_ref, b_ref, o_ref):
        doc = s_ref[...] * (1.0 / SEQ)
        o_ref[...] = (
            jnp.dot(doc, w_ref[...], preferred_element_type=jnp.float32) + b_ref[...]
        )

    return pl.pallas_call(
        head,
        out_shape=jax.ShapeDtypeStruct((BATCH, NUM_CLASSES), jnp.float32),
    )(sums, wt, bias)


def kernel(x, table, W, b):
    x_r = x.astype(jnp.int32).reshape(NW, BPW, NCHUNK, CHUNK)
    sums = _sc_pool(x_r, table)
    return _tc_head(sums.reshape(BATCH, EMBED), W.T, b.reshape(1, NUM_CLASSES))
